# initial kernel scaffold (unmeasured)
import jax
import jax.numpy as jnp
from jax import lax
from jax.experimental import pallas as pl
from jax.experimental.pallas import tpu as pltpu


def kernel(
    x,
):
    def body(*refs):
        pass

    out_shape = jax.ShapeDtypeStruct(..., jnp.float32)
    return pl.pallas_call(body, out_shape=out_shape)(...)


# baseline (device time: 13401 ns/iter reference)
import jax
import jax.numpy as jnp
from jax import lax
from jax.experimental import pallas as pl
from jax.experimental.pallas import tpu as pltpu

N_DEV = 16


def kernel(x):
    m, n = x.shape

    def body(x_ref, out_ref, halo_ref, send_sem, recv_sem):
        my = lax.axis_index("i")
        has_left = my > 0
        has_right = my < N_DEV - 1


        @pl.when(has_left)
        def _():
            send_l = pltpu.make_async_remote_copy(
                src_ref=x_ref.at[pl.ds(0, 1)],
                dst_ref=halo_ref.at[pl.ds(1, 1)],
                send_sem=send_sem.at[0],
                recv_sem=recv_sem.at[1],
                device_id=(my - 1,),
                device_id_type=pl.DeviceIdType.MESH,
            )
            send_l.start()
            send_l.wait_send()

        @pl.when(has_right)
        def _():
            send_r = pltpu.make_async_remote_copy(
                src_ref=x_ref.at[pl.ds(m - 1, 1)],
                dst_ref=halo_ref.at[pl.ds(0, 1)],
                send_sem=send_sem.at[1],
                recv_sem=recv_sem.at[0],
                device_id=(my + 1,),
                device_id_type=pl.DeviceIdType.MESH,
            )
            send_r.start()
            send_r.wait_send()

        @pl.when(has_left)
        def _():
            recv_top = pltpu.make_async_remote_copy(
                src_ref=x_ref.at[pl.ds(0, 1)],
                dst_ref=halo_ref.at[pl.ds(0, 1)],
                send_sem=send_sem.at[0],
                recv_sem=recv_sem.at[0],
                device_id=(my - 1,),
                device_id_type=pl.DeviceIdType.MESH,
            )
            recv_top.wait_recv()

        @pl.when(has_right)
        def _():
            recv_bot = pltpu.make_async_remote_copy(
                src_ref=x_ref.at[pl.ds(0, 1)],
                dst_ref=halo_ref.at[pl.ds(1, 1)],
                send_sem=send_sem.at[1],
                recv_sem=recv_sem.at[1],
                device_id=(my + 1,),
                device_id_type=pl.DeviceIdType.MESH,
            )
            recv_bot.wait_recv()

        xv = x_ref[:]
        top = halo_ref[pl.ds(0, 1)]
        bot = halo_ref[pl.ds(1, 1)]
        xm = jnp.concatenate([top, xv, bot], axis=0)
        res = 0.25 * xm[:-2] + 0.5 * xm[1:-1] + 0.25 * xm[2:]
        row0 = jnp.where(has_left, res[0:1], xv[0:1])
        rowl = jnp.where(has_right, res[m - 1 : m], xv[m - 1 : m])
        out_ref[:] = jnp.concatenate([row0, res[1 : m - 1], rowl], axis=0)

    return pl.pallas_call(
        body,
        out_shape=jax.ShapeDtypeStruct((m, n), x.dtype),
        in_specs=[pl.BlockSpec(memory_space=pltpu.VMEM)],
        out_specs=pl.BlockSpec(memory_space=pltpu.VMEM),
        scratch_shapes=[
            pltpu.VMEM((2, n), x.dtype),
            pltpu.SemaphoreType.DMA((2,)),
            pltpu.SemaphoreType.DMA((2,)),
        ],
    )(x)


# device time: 7457 ns/iter; 1.7971x vs baseline; 1.7971x over previous
import jax
import jax.numpy as jnp
from jax import lax
from jax.experimental import pallas as pl
from jax.experimental.pallas import tpu as pltpu

N_DEV = 16


def kernel(x):
    m, n = x.shape

    def body(x_ref, out_ref, halo_ref, send_sem, recv_sem):
        my = lax.axis_index("i")
        has_left = my > 0
        has_right = my < N_DEV - 1

        barrier = pltpu.get_barrier_semaphore()

        @pl.when(has_left)
        def _():
            pl.semaphore_signal(
                barrier, inc=1, device_id=(my - 1,),
                device_id_type=pl.DeviceIdType.MESH,
            )

        @pl.when(has_right)
        def _():
            pl.semaphore_signal(
                barrier, inc=1, device_id=(my + 1,),
                device_id_type=pl.DeviceIdType.MESH,
            )

        n_nbrs = has_left.astype(jnp.int32) + has_right.astype(jnp.int32)
        pl.semaphore_wait(barrier, n_nbrs)

        send_l = pltpu.make_async_remote_copy(
            src_ref=x_ref.at[pl.ds(0, 1)],
            dst_ref=halo_ref.at[pl.ds(1, 1)],
            send_sem=send_sem.at[0],
            recv_sem=recv_sem.at[1],
            device_id=(my - 1,),
            device_id_type=pl.DeviceIdType.MESH,
        )
        send_r = pltpu.make_async_remote_copy(
            src_ref=x_ref.at[pl.ds(m - 1, 1)],
            dst_ref=halo_ref.at[pl.ds(0, 1)],
            send_sem=send_sem.at[1],
            recv_sem=recv_sem.at[0],
            device_id=(my + 1,),
            device_id_type=pl.DeviceIdType.MESH,
        )

        @pl.when(has_left)
        def _():
            send_l.start()

        @pl.when(has_right)
        def _():
            send_r.start()

        xv = x_ref[:]
        xm = jnp.concatenate([xv[0:1], xv, xv[m - 1 : m]], axis=0)
        res = 0.25 * xm[:-2] + 0.5 * xm[1:-1] + 0.25 * xm[2:]
        out_ref[:] = res

        @pl.when(has_left)
        def _():
            recv_top = pltpu.make_async_remote_copy(
                src_ref=x_ref.at[pl.ds(0, 1)],
                dst_ref=halo_ref.at[pl.ds(0, 1)],
                send_sem=send_sem.at[0],
                recv_sem=recv_sem.at[0],
                device_id=(my - 1,),
                device_id_type=pl.DeviceIdType.MESH,
            )
            recv_top.wait_recv()

        @pl.when(has_right)
        def _():
            recv_bot = pltpu.make_async_remote_copy(
                src_ref=x_ref.at[pl.ds(0, 1)],
                dst_ref=halo_ref.at[pl.ds(1, 1)],
                send_sem=send_sem.at[1],
                recv_sem=recv_sem.at[1],
                device_id=(my + 1,),
                device_id_type=pl.DeviceIdType.MESH,
            )
            recv_bot.wait_recv()

        top = halo_ref[pl.ds(0, 1)]
        bot = halo_ref[pl.ds(1, 1)]
        row0 = jnp.where(
            has_left, 0.25 * top + 0.5 * xv[0:1] + 0.25 * xv[1:2], xv[0:1]
        )
        rowl = jnp.where(
            has_right,
            0.25 * xv[m - 2 : m - 1] + 0.5 * xv[m - 1 : m] + 0.25 * bot,
            xv[m - 1 : m],
        )
        out_ref[pl.ds(0, 1)] = row0
        out_ref[pl.ds(m - 1, 1)] = rowl

        @pl.when(has_left)
        def _():
            send_l.wait_send()

        @pl.when(has_right)
        def _():
            send_r.wait_send()

    return pl.pallas_call(
        body,
        out_shape=jax.ShapeDtypeStruct((m, n), x.dtype),
        in_specs=[pl.BlockSpec(memory_space=pltpu.VMEM)],
        out_specs=pl.BlockSpec(memory_space=pltpu.VMEM),
        scratch_shapes=[
            pltpu.VMEM((2, n), x.dtype),
            pltpu.SemaphoreType.DMA((2,)),
            pltpu.SemaphoreType.DMA((2,)),
        ],
        compiler_params=pltpu.CompilerParams(collective_id=0),
    )(x)


# device time: 7435 ns/iter; 1.8024x vs baseline; 1.0030x over previous
import jax
import jax.numpy as jnp
from jax import lax
from jax.experimental import pallas as pl
from jax.experimental.pallas import tpu as pltpu

N_DEV = 16


def kernel(x):
    m, n = x.shape

    def body(x_ref, out_ref, halo_ref, send_sem, recv_sem):
        my = lax.axis_index("i")
        has_left = my > 0
        has_right = my < N_DEV - 1

        barrier = pltpu.get_barrier_semaphore()

        @pl.when(has_left)
        def _():
            pl.semaphore_signal(
                barrier, inc=1, device_id=(my - 1,),
                device_id_type=pl.DeviceIdType.MESH,
            )

        @pl.when(has_right)
        def _():
            pl.semaphore_signal(
                barrier, inc=1, device_id=(my + 1,),
                device_id_type=pl.DeviceIdType.MESH,
            )

        n_nbrs = has_left.astype(jnp.int32) + has_right.astype(jnp.int32)
        pl.semaphore_wait(barrier, n_nbrs)

        send_l = pltpu.make_async_remote_copy(
            src_ref=x_ref.at[pl.ds(0, 1)],
            dst_ref=halo_ref.at[pl.ds(1, 1)],
            send_sem=send_sem.at[0],
            recv_sem=recv_sem.at[1],
            device_id=(my - 1,),
            device_id_type=pl.DeviceIdType.MESH,
        )
        send_r = pltpu.make_async_remote_copy(
            src_ref=x_ref.at[pl.ds(m - 1, 1)],
            dst_ref=halo_ref.at[pl.ds(0, 1)],
            send_sem=send_sem.at[1],
            recv_sem=recv_sem.at[0],
            device_id=(my + 1,),
            device_id_type=pl.DeviceIdType.MESH,
        )

        @pl.when(has_left)
        def _():
            send_l.start()

        @pl.when(has_right)
        def _():
            send_r.start()

        xv = x_ref[:]
        res_mid = 0.25 * (xv[: m - 2] + xv[2:]) + 0.5 * xv[1 : m - 1]
        out_ref[pl.ds(1, m - 2)] = res_mid

        @pl.when(has_left)
        def _():
            recv_top = pltpu.make_async_remote_copy(
                src_ref=x_ref.at[pl.ds(0, 1)],
                dst_ref=halo_ref.at[pl.ds(0, 1)],
                send_sem=send_sem.at[0],
                recv_sem=recv_sem.at[0],
                device_id=(my - 1,),
                device_id_type=pl.DeviceIdType.MESH,
            )
            recv_top.wait_recv()

        @pl.when(has_right)
        def _():
            recv_bot = pltpu.make_async_remote_copy(
                src_ref=x_ref.at[pl.ds(0, 1)],
                dst_ref=halo_ref.at[pl.ds(1, 1)],
                send_sem=send_sem.at[1],
                recv_sem=recv_sem.at[1],
                device_id=(my + 1,),
                device_id_type=pl.DeviceIdType.MESH,
            )
            recv_bot.wait_recv()

        top = halo_ref[pl.ds(0, 1)]
        bot = halo_ref[pl.ds(1, 1)]
        row0 = jnp.where(
            has_left, 0.25 * top + 0.5 * xv[0:1] + 0.25 * xv[1:2], xv[0:1]
        )
        rowl = jnp.where(
            has_right,
            0.25 * xv[m - 2 : m - 1] + 0.5 * xv[m - 1 : m] + 0.25 * bot,
            xv[m - 1 : m],
        )
        out_ref[pl.ds(0, 1)] = row0
        out_ref[pl.ds(m - 1, 1)] = rowl

        @pl.when(has_left)
        def _():
            send_l.wait_send()

        @pl.when(has_right)
        def _():
            send_r.wait_send()

    return pl.pallas_call(
        body,
        out_shape=jax.ShapeDtypeStruct((m, n), x.dtype),
        in_specs=[pl.BlockSpec(memory_space=pltpu.VMEM)],
        out_specs=pl.BlockSpec(memory_space=pltpu.VMEM),
        scratch_shapes=[
            pltpu.VMEM((2, n), x.dtype),
            pltpu.SemaphoreType.DMA((2,)),
            pltpu.SemaphoreType.DMA((2,)),
        ],
        compiler_params=pltpu.CompilerParams(collective_id=0),
    )(x)
